# baseline (speedup 1.0000x reference)
import functools

import jax
import jax.numpy as jnp
from jax import lax
from jax.experimental import pallas as pl
from jax.experimental.pallas import tpu as pltpu, tpu_sc as plsc


def _make_sc_gather(V, D, B):
    info = plsc.get_sparse_core_info()
    NW = info.num_cores * info.num_subcores
    assert B % (8 * NW) == 0 and D % info.num_lanes == 0
    b_per_w = B // NW
    mesh = plsc.VectorSubcoreMesh(core_axis_name="c", subcore_axis_name="s")

    @functools.partial(
        pl.kernel,
        mesh=mesh,
        compiler_params=pltpu.CompilerParams(use_tc_tiling_on_sc=False),
        out_type=jax.ShapeDtypeStruct((B, D), jnp.float32),
        scratch_types=[
            pltpu.VMEM((b_per_w,), jnp.int32),
            pltpu.VMEM((b_per_w, D), jnp.float32),
            pltpu.SemaphoreType.DMA,
        ],
    )
    def emb(labels_hbm, table_hbm, out_hbm, idx_v, rows_v, sem):
        wid = lax.axis_index("s") * info.num_cores + lax.axis_index("c")
        base = wid * b_per_w
        pltpu.sync_copy(labels_hbm.at[pl.ds(base, b_per_w)], idx_v)
        pltpu.async_copy(table_hbm.at[idx_v], rows_v, sem).wait()
        pltpu.sync_copy(rows_v, out_hbm.at[pl.ds(base, b_per_w)])

    return emb


def kernel(labels, embedding_table):
    B = labels.shape[0]
    V, D = embedding_table.shape
    emb = _make_sc_gather(V, D, B)
    return emb(labels.astype(jnp.int32), embedding_table)


# speedup vs baseline: 1.6748x; 1.6748x over previous
import functools

import jax
import jax.numpy as jnp
from jax import lax
from jax.experimental import pallas as pl
from jax.experimental.pallas import tpu as pltpu, tpu_sc as plsc


def _make_sc_gather(V, D, B):
    info = plsc.get_sparse_core_info()
    NW = info.num_cores * info.num_subcores
    assert B % (8 * NW) == 0 and D % info.num_lanes == 0
    b_per_w = B // NW
    K = 16
    n_chunks = b_per_w // K
    mesh = plsc.VectorSubcoreMesh(core_axis_name="c", subcore_axis_name="s")

    @functools.partial(
        pl.kernel,
        mesh=mesh,
        out_type=jax.ShapeDtypeStruct((B, D), jnp.float32),
        scratch_types=[
            pltpu.VMEM((b_per_w,), jnp.int32),
            pltpu.VMEM((b_per_w, D), jnp.float32),
            pltpu.SemaphoreType.DMA,
        ],
    )
    def emb(labels_hbm, table_hbm, out_hbm, idx_v, rows_v, sem):
        wid = lax.axis_index("s") * info.num_cores + lax.axis_index("c")
        base = wid * b_per_w
        pltpu.sync_copy(labels_hbm.at[pl.ds(base, b_per_w)], idx_v)

        def fire(c):
            vec = idx_v[pl.ds(c * K, K)]
            for j in range(K):
                row = vec[j]
                pltpu.async_copy(
                    table_hbm.at[pl.ds(row, 1)],
                    rows_v.at[pl.ds(c * K + j, 1)],
                    sem,
                )

        def drain():
            for _ in range(K):
                pltpu.make_async_copy(
                    table_hbm.at[pl.ds(0, 1)], rows_v.at[pl.ds(0, 1)], sem
                ).wait()

        fire(0)

        def body(c, _):
            fire(c)
            drain()
            return 0

        lax.fori_loop(1, n_chunks, body, 0)
        drain()
        pltpu.sync_copy(rows_v, out_hbm.at[pl.ds(base, b_per_w)])

    return emb


def kernel(labels, embedding_table):
    B = labels.shape[0]
    V, D = embedding_table.shape
    emb = _make_sc_gather(V, D, B)
    return emb(labels.astype(jnp.int32), embedding_table)


# speedup vs baseline: 1.7055x; 1.0184x over previous
import functools

import jax
import jax.numpy as jnp
from jax import lax
from jax.experimental import pallas as pl
from jax.experimental.pallas import tpu as pltpu, tpu_sc as plsc


def _make_sc_gather(V, D, B, depth_chunks=4):
    info = plsc.get_sparse_core_info()
    L = info.num_lanes
    NW = info.num_cores * info.num_subcores
    assert B % (8 * NW) == 0 and D % L == 0
    b_per_w = B // NW
    n_chunks = b_per_w // L
    mesh = plsc.VectorSubcoreMesh(core_axis_name="c", subcore_axis_name="s")

    @functools.partial(
        pl.kernel,
        mesh=mesh,
        out_type=jax.ShapeDtypeStruct((B, D), jnp.float32),
        scratch_types=[
            pltpu.VMEM((b_per_w,), jnp.int32),
            pltpu.VMEM((b_per_w, D), jnp.float32),
            pltpu.SemaphoreType.DMA,
        ],
    )
    def emb(labels_hbm, table_hbm, out_hbm, idx_v, rows_v, sem):
        wid = lax.axis_index("s") * info.num_cores + lax.axis_index("c")
        base = wid * b_per_w
        pltpu.sync_copy(labels_hbm.at[pl.ds(base, b_per_w)], idx_v)

        def fire(c):
            vec = idx_v[pl.ds(c * L, L)]
            for j in range(L):
                row = vec[j]
                pltpu.async_copy(
                    table_hbm.at[pl.ds(row, 1)],
                    rows_v.at[pl.ds(c * L + j, 1)],
                    sem,
                )

        def drain():
            for _ in range(L):
                pltpu.make_async_copy(
                    table_hbm.at[pl.ds(0, 1)], rows_v.at[pl.ds(0, 1)], sem
                ).wait()

        for c in range(depth_chunks):
            fire(c)

        def body(c, _):
            fire(c)
            drain()
            return 0

        lax.fori_loop(depth_chunks, n_chunks, body, 0)
        for _ in range(depth_chunks):
            drain()
        pltpu.sync_copy(rows_v, out_hbm.at[pl.ds(base, b_per_w)])

    return emb


def kernel(labels, embedding_table):
    B = labels.shape[0]
    V, D = embedding_table.shape
    emb = _make_sc_gather(V, D, B)
    return emb(labels.astype(jnp.int32), embedding_table)
